# baseline (speedup 1.0000x reference)
import numpy as np
import jax
import jax.numpy as jnp
from jax.experimental import pallas as pl
from jax.experimental.pallas import tpu as pltpu

_B = 512

_F32 = jnp.float32

_IY = np.arange(4)[None, :, None]
_YY = np.arange(2)[None, None, :]
_DY = np.arange(3)[:, None, None]
_P_GEN = (_IY - _YY == _DY).astype(np.float32)
_P_TOP = (_IY - _YY + 1 == _DY).astype(np.float32)
_P_BOT = (_IY - _YY - 1 == _DY).astype(np.float32)
_Q1 = (np.arange(28)[None, :, None, None]
       - 2 * np.arange(14)[None, None, None, :]
       - np.arange(2)[None, None, :, None] + 1
       == np.arange(3)[:, None, None, None]).astype(np.float32)
_Q2 = (np.arange(14)[None, :, None, None]
       - 2 * np.arange(7)[None, None, None, :]
       - np.arange(2)[None, None, :, None] + 1
       == np.arange(3)[:, None, None, None]).astype(np.float32)


def _c1_band(p, w1t):
    w = jnp.einsum("aIY,bXPW,abc->IXYPcW", p, _Q1, w1t)
    w = jnp.pad(w.reshape(112, 2, 2, 224), ((0, 0), (0, 0), (0, 0), (0, 32)))
    return w.reshape(112, 1024)


def _prep_mats(w1_taps, b1, w2_taps, b2, fc1_mat):
    w1t = w1_taps.reshape(3, 3, 16)
    w2t = w2_taps.reshape(3, 3, 16, 32)
    w1g, w1top, w1bot = (_c1_band(p, w1t) for p in (_P_GEN, _P_TOP, _P_BOT))
    w2band = jnp.einsum("bXPW,abio->aiXPoW", _Q2, w2t)
    w2band = w2band.reshape(3, 224, 2, 224)
    w2band = jnp.pad(w2band, ((0, 0), (0, 32), (0, 0), (0, 32)))
    w2band = w2band.reshape(768, 512)
    fc1p = fc1_mat.reshape(7, 7, 32, 128).transpose(0, 2, 1, 3)
    fc1p = jnp.pad(fc1p.reshape(7, 224, 128), ((0, 0), (0, 32), (0, 0)))
    fc1p = fc1p.reshape(1792, 128)
    b1v = jnp.pad(jnp.repeat(b1.reshape(16, 1), 14, axis=1).reshape(1, 224),
                  ((0, 0), (0, 32)))
    b2v = jnp.pad(jnp.repeat(b2.reshape(32, 1), 7, axis=1).reshape(1, 224),
                  ((0, 0), (0, 32)))
    return w1g, w1top, w1bot, w2band, fc1p, b1v, b2v


def _fused_kernel(x_ref, w1g_ref, w1t_ref, w1b_ref, b1_ref, w2b_ref, b2_ref,
                  fc1_ref, fc1b_ref, fc2_ref, fc2b_ref, o_ref):
    X = x_ref[...]
    b1v = b1_ref[...]
    w2b = w2b_ref[...]
    b2v = b2_ref[...]

    B = X.shape[0]
    zrow = jnp.zeros((B, 256), _F32)

    p1rows = [zrow]
    for h in range(14):
        if h == 0:
            sl, wb = X[:, 0:112], w1t_ref[...]
        elif h == 13:
            sl, wb = X[:, 672:784], w1b_ref[...]
        else:
            sl, wb = X[:, 56 * h - 28:56 * h + 84], w1g_ref[...]
        u = jnp.dot(sl, wb, preferred_element_type=_F32)
        v = jnp.maximum(u[:, :512], u[:, 512:])
        vh = jnp.maximum(v[:, :256], v[:, 256:])
        p1rows.append(jnp.maximum(vh + b1v, 0.0))
    p1rows.append(zrow)
    P1 = jnp.concatenate(p1rows, axis=1)

    urows = [jnp.dot(P1[:, 256 * h:256 * h + 768], w2b,
                     preferred_element_type=_F32)
             for h in range(14)]
    p2rows = []
    for h in range(7):
        v = jnp.maximum(urows[2 * h], urows[2 * h + 1])
        vh = jnp.maximum(v[:, :256], v[:, 256:])
        p2rows.append(jnp.maximum(vh + b2v, 0.0))
    P2 = jnp.concatenate(p2rows, axis=1)

    hmid = jnp.dot(P2, fc1_ref[...], preferred_element_type=_F32)
    hmid = jnp.maximum(hmid + fc1b_ref[...], 0.0)
    o_ref[...] = (jnp.dot(hmid, fc2_ref[...], preferred_element_type=_F32)
                  + fc2b_ref[...])


def kernel(x_nchw, w1_taps, b1, w2_taps, b2, s1, s2,
           fc1_mat, fc1_b, fc2_mat, fc2_b):
    del s1, s2
    N = x_nchw.shape[0]
    Np = (N + _B - 1) // _B * _B
    x = x_nchw.astype(_F32).reshape(N, 784)
    if Np != N:
        x = jnp.pad(x, ((0, Np - N), (0, 0)))

    w1g, w1top, w1bot, w2band, fc1p, b1v, b2v = _prep_mats(
        w1_taps, b1, w2_taps, b2, fc1_mat)

    out = pl.pallas_call(
        _fused_kernel,
        out_shape=jax.ShapeDtypeStruct((Np, 10), _F32),
        grid=(Np // _B,),
        in_specs=[
            pl.BlockSpec((_B, 784), lambda i: (i, 0)),
            pl.BlockSpec((112, 1024), lambda i: (0, 0)),
            pl.BlockSpec((112, 1024), lambda i: (0, 0)),
            pl.BlockSpec((112, 1024), lambda i: (0, 0)),
            pl.BlockSpec((1, 256), lambda i: (0, 0)),
            pl.BlockSpec((768, 512), lambda i: (0, 0)),
            pl.BlockSpec((1, 256), lambda i: (0, 0)),
            pl.BlockSpec((1792, 128), lambda i: (0, 0)),
            pl.BlockSpec((1, 128), lambda i: (0, 0)),
            pl.BlockSpec((128, 10), lambda i: (0, 0)),
            pl.BlockSpec((1, 10), lambda i: (0, 0)),
        ],
        out_specs=pl.BlockSpec((_B, 10), lambda i: (i, 0)),
        compiler_params=pltpu.CompilerParams(
            dimension_semantics=("parallel",),
            vmem_limit_bytes=64 * 1024 * 1024),
    )(x, w1g, w1top, w1bot, b1v, w2band, b2v, fc1p, fc1_b, fc2_mat, fc2_b)
    return out[:N]


# speedup vs baseline: 1.1771x; 1.1771x over previous
import numpy as np
import jax
import jax.numpy as jnp
from jax.experimental import pallas as pl
from jax.experimental.pallas import tpu as pltpu

_B = 512

_F32 = jnp.float32

_PY = (np.arange(4)[None, :, None] - np.arange(2)[None, None, :]
       == np.arange(3)[:, None, None]).astype(np.float32)
_Q1 = (np.arange(32)[None, :, None, None]
       - 2 * np.arange(14)[None, None, None, :]
       - np.arange(2)[None, None, :, None]
       == np.arange(3)[:, None, None, None]).astype(np.float32)
_Q2 = (np.arange(14)[None, :, None, None]
       - 2 * np.arange(7)[None, None, None, :]
       - np.arange(2)[None, None, :, None] + 1
       == np.arange(3)[:, None, None, None]).astype(np.float32)


def _prep_mats(w1_taps, b1, w2_taps, b2, fc1_mat):
    w1t = w1_taps.reshape(3, 3, 16)
    w2t = w2_taps.reshape(3, 3, 16, 32)
    w1band = jnp.einsum("aIY,bXPW,abc->IXYPcW", _PY, _Q1, w1t)
    w1band = w1band.reshape(128, 2, 2, 224)
    w1band = jnp.pad(w1band, ((0, 0), (0, 0), (0, 0), (0, 32)))
    w1band = w1band.reshape(128, 1024)
    w2band = jnp.einsum("bXPW,abio->aiXPoW", _Q2, w2t)
    w2band = w2band.reshape(3, 224, 2, 224)
    w2band = jnp.pad(w2band, ((0, 0), (0, 32), (0, 0), (0, 32)))
    w2band = w2band.reshape(768, 512)
    fc1p = fc1_mat.reshape(7, 7, 32, 128).transpose(0, 2, 1, 3)
    fc1p = jnp.pad(fc1p.reshape(7, 224, 128), ((0, 0), (0, 32), (0, 0)))
    fc1p = fc1p.reshape(1792, 128)
    b1v = jnp.pad(jnp.repeat(b1.reshape(16, 1), 14, axis=1).reshape(1, 224),
                  ((0, 0), (0, 32)))
    b2v = jnp.pad(jnp.repeat(b2.reshape(32, 1), 7, axis=1).reshape(1, 224),
                  ((0, 0), (0, 32)))
    return w1band, w2band, fc1p, b1v, b2v


def _fused_kernel(x_ref, w1b_ref, b1_ref, w2b_ref, b2_ref,
                  fc1_ref, fc1b_ref, fc2_ref, fc2b_ref, o_ref):
    X = x_ref[...]
    w1b = w1b_ref[...]
    b1v = b1_ref[...]
    w2b = w2b_ref[...]
    b2v = b2_ref[...]

    B = X.shape[0]
    zrow = jnp.zeros((B, 256), _F32)

    p1rows = [zrow]
    for h in range(14):
        u = jnp.dot(X[:, 64 * h:64 * h + 128], w1b,
                    preferred_element_type=_F32)
        v = jnp.maximum(u[:, :512], u[:, 512:])
        vh = jnp.maximum(v[:, :256], v[:, 256:])
        p1rows.append(jnp.maximum(vh + b1v, 0.0))
    p1rows.append(zrow)
    P1 = jnp.concatenate(p1rows, axis=1)

    urows = [jnp.dot(P1[:, 256 * h:256 * h + 768], w2b,
                     preferred_element_type=_F32)
             for h in range(14)]
    p2rows = []
    for h in range(7):
        v = jnp.maximum(urows[2 * h], urows[2 * h + 1])
        vh = jnp.maximum(v[:, :256], v[:, 256:])
        p2rows.append(jnp.maximum(vh + b2v, 0.0))
    P2 = jnp.concatenate(p2rows, axis=1)

    hmid = jnp.dot(P2, fc1_ref[...], preferred_element_type=_F32)
    hmid = jnp.maximum(hmid + fc1b_ref[...], 0.0)
    o_ref[...] = (jnp.dot(hmid, fc2_ref[...], preferred_element_type=_F32)
                  + fc2b_ref[...])


def kernel(x_nchw, w1_taps, b1, w2_taps, b2, s1, s2,
           fc1_mat, fc1_b, fc2_mat, fc2_b):
    del s1, s2
    N = x_nchw.shape[0]
    Np = (N + _B - 1) // _B * _B
    x = x_nchw.astype(_F32)[:, 0, :, :]
    x = jnp.pad(x, ((0, Np - N), (1, 1), (1, 3))).reshape(Np, 960)

    w1band, w2band, fc1p, b1v, b2v = _prep_mats(w1_taps, b1, w2_taps, b2,
                                                fc1_mat)

    out = pl.pallas_call(
        _fused_kernel,
        out_shape=jax.ShapeDtypeStruct((Np, 10), _F32),
        grid=(Np // _B,),
        in_specs=[
            pl.BlockSpec((_B, 960), lambda i: (i, 0)),
            pl.BlockSpec((128, 1024), lambda i: (0, 0)),
            pl.BlockSpec((1, 256), lambda i: (0, 0)),
            pl.BlockSpec((768, 512), lambda i: (0, 0)),
            pl.BlockSpec((1, 256), lambda i: (0, 0)),
            pl.BlockSpec((1792, 128), lambda i: (0, 0)),
            pl.BlockSpec((1, 128), lambda i: (0, 0)),
            pl.BlockSpec((128, 10), lambda i: (0, 0)),
            pl.BlockSpec((1, 10), lambda i: (0, 0)),
        ],
        out_specs=pl.BlockSpec((_B, 10), lambda i: (i, 0)),
        compiler_params=pltpu.CompilerParams(
            dimension_semantics=("parallel",),
            vmem_limit_bytes=64 * 1024 * 1024),
    )(x, w1band, b1v, w2band, b2v, fc1p, fc1_b, fc2_mat, fc2_b)
    return out[:N]


# speedup vs baseline: 1.1814x; 1.0036x over previous
import numpy as np
import jax
import jax.numpy as jnp
from jax.experimental import pallas as pl
from jax.experimental.pallas import tpu as pltpu

_B = 1024

_F32 = jnp.float32

_PY = (np.arange(4)[None, :, None] - np.arange(2)[None, None, :]
       == np.arange(3)[:, None, None]).astype(np.float32)
_Q1 = (np.arange(32)[None, :, None, None]
       - 2 * np.arange(14)[None, None, None, :]
       - np.arange(2)[None, None, :, None]
       == np.arange(3)[:, None, None, None]).astype(np.float32)
_Q2 = (np.arange(14)[None, :, None, None]
       - 2 * np.arange(7)[None, None, None, :]
       - np.arange(2)[None, None, :, None] + 1
       == np.arange(3)[:, None, None, None]).astype(np.float32)


def _prep_mats(w1_taps, b1, w2_taps, b2, fc1_mat):
    w1t = w1_taps.reshape(3, 3, 16)
    w2t = w2_taps.reshape(3, 3, 16, 32)
    w1band = jnp.einsum("aIY,bXPW,abc->IXYPcW", _PY, _Q1, w1t)
    w1band = w1band.reshape(128, 2, 2, 224)
    w1band = jnp.pad(w1band, ((0, 0), (0, 0), (0, 0), (0, 32)))
    w1band = w1band.reshape(128, 1024)
    w2band = jnp.einsum("bXPW,abio->aiXPoW", _Q2, w2t)
    w2band = w2band.reshape(3, 224, 2, 224)
    w2band = jnp.pad(w2band, ((0, 0), (0, 32), (0, 0), (0, 32)))
    w2band = w2band.reshape(768, 512)
    fc1p = fc1_mat.reshape(7, 7, 32, 128).transpose(0, 2, 1, 3)
    fc1p = jnp.pad(fc1p.reshape(7, 224, 128), ((0, 0), (0, 32), (0, 0)))
    fc1p = fc1p.reshape(1792, 128)
    b1v = jnp.pad(jnp.repeat(b1.reshape(16, 1), 14, axis=1).reshape(1, 224),
                  ((0, 0), (0, 32)))
    b2v = jnp.pad(jnp.repeat(b2.reshape(32, 1), 7, axis=1).reshape(1, 224),
                  ((0, 0), (0, 32)))
    return w1band, w2band, fc1p, b1v, b2v


def _fused_kernel(x_ref, w1b_ref, b1_ref, w2b_ref, b2_ref,
                  fc1_ref, fc1b_ref, fc2_ref, fc2b_ref, o_ref):
    X = x_ref[...]
    w1b = w1b_ref[...]
    b1v = b1_ref[...]
    w2b = w2b_ref[...]
    b2v = b2_ref[...]

    B = X.shape[0]
    zrow = jnp.zeros((B, 256), _F32)

    p1rows = [zrow]
    for h in range(14):
        u = jnp.dot(X[:, 64 * h:64 * h + 128], w1b,
                    preferred_element_type=_F32)
        v = jnp.maximum(u[:, :512], u[:, 512:])
        vh = jnp.maximum(v[:, :256], v[:, 256:])
        p1rows.append(jnp.maximum(vh + b1v, 0.0))
    p1rows.append(zrow)
    P1 = jnp.concatenate(p1rows, axis=1)

    urows = [jnp.dot(P1[:, 256 * h:256 * h + 768], w2b,
                     preferred_element_type=_F32)
             for h in range(14)]
    p2rows = []
    for h in range(7):
        v = jnp.maximum(urows[2 * h], urows[2 * h + 1])
        vh = jnp.maximum(v[:, :256], v[:, 256:])
        p2rows.append(jnp.maximum(vh + b2v, 0.0))
    P2 = jnp.concatenate(p2rows, axis=1)

    hmid = jnp.dot(P2, fc1_ref[...], preferred_element_type=_F32)
    hmid = jnp.maximum(hmid + fc1b_ref[...], 0.0)
    o_ref[...] = (jnp.dot(hmid, fc2_ref[...], preferred_element_type=_F32)
                  + fc2b_ref[...])


def kernel(x_nchw, w1_taps, b1, w2_taps, b2, s1, s2,
           fc1_mat, fc1_b, fc2_mat, fc2_b):
    del s1, s2
    N = x_nchw.shape[0]
    Np = (N + _B - 1) // _B * _B
    x = x_nchw.astype(_F32)[:, 0, :, :]
    x = jnp.pad(x, ((0, Np - N), (1, 1), (1, 3))).reshape(Np, 960)

    w1band, w2band, fc1p, b1v, b2v = _prep_mats(w1_taps, b1, w2_taps, b2,
                                                fc1_mat)

    out = pl.pallas_call(
        _fused_kernel,
        out_shape=jax.ShapeDtypeStruct((Np, 10), _F32),
        grid=(Np // _B,),
        in_specs=[
            pl.BlockSpec((_B, 960), lambda i: (i, 0)),
            pl.BlockSpec((128, 1024), lambda i: (0, 0)),
            pl.BlockSpec((1, 256), lambda i: (0, 0)),
            pl.BlockSpec((768, 512), lambda i: (0, 0)),
            pl.BlockSpec((1, 256), lambda i: (0, 0)),
            pl.BlockSpec((1792, 128), lambda i: (0, 0)),
            pl.BlockSpec((1, 128), lambda i: (0, 0)),
            pl.BlockSpec((128, 10), lambda i: (0, 0)),
            pl.BlockSpec((1, 10), lambda i: (0, 0)),
        ],
        out_specs=pl.BlockSpec((_B, 10), lambda i: (i, 0)),
        compiler_params=pltpu.CompilerParams(
            dimension_semantics=("parallel",),
            vmem_limit_bytes=64 * 1024 * 1024),
    )(x, w1band, b1v, w2band, b2v, fc1p, fc1_b, fc2_mat, fc2_b)
    return out[:N]
